# baseline (speedup 1.0000x reference)
import functools

import jax
import jax.numpy as jnp
from jax import lax
from jax.experimental import pallas as pl
from jax.experimental.pallas import tpu as pltpu
from jax.experimental.pallas import tpu_sc as plsc

_N = 6
_D = 32
_L = 16
_STEPS = _N - 1

_ROW_X = 0
_ROW_WPT = 12
_ROW_BP = 44
_ROW_G = 45
_ROWS = 50


def _energy_body(x_ref, w1_ref, b1_ref, w2_ref, b2_ref, w3_ref, b3_ref, o_ref):
    x = x_ref[...]
    h = jnp.dot(x, w1_ref[...], preferred_element_type=jnp.float32)
    h = jnp.maximum(h + b1_ref[...], 0.0)
    h = jnp.dot(h, w2_ref[...], preferred_element_type=jnp.float32)
    h = jnp.maximum(h + b2_ref[...], 0.0)
    z = jnp.dot(h, w3_ref[...], preferred_element_type=jnp.float32)
    z = z + b3_ref[...]
    o_ref[...] = jnp.maximum(z, 0.0) + jnp.log(1.0 + jnp.exp(-jnp.abs(z)))


def _energy_head(x8, w1t, b1r, w2t, b2r, w3t, b3r):
    return pl.pallas_call(
        _energy_body,
        out_shape=jax.ShapeDtypeStruct((8, 8), jnp.float32),
    )(x8, w1t, b1r, w2t, b2r, w3t, b3r)


def _lane_bcast(vec, lane):
    idx = jnp.full((_L,), lane, dtype=jnp.int32)
    return jnp.take_along_axis(vec, idx, axis=0, mode="promise_in_bounds")


def _sample_body(blk_hbm, out_hbm, blk_v, l_v, path_v):
    cid = lax.axis_index("c")
    sid = lax.axis_index("s")

    @pl.when(jnp.logical_and(cid == 0, sid == 0))
    def _():
        pltpu.sync_copy(blk_hbm, blk_v)

        bp_vec = blk_v[_ROW_BP, :]
        w_rows = [blk_v[_ROW_WPT + k, :] for k in range(_D)]
        rows = []
        for i in range(_N):
            xia = blk_v[_ROW_X + 2 * i, :]
            xib = blk_v[_ROW_X + 2 * i + 1, :]
            acc = bp_vec
            for k in range(_D):
                xk = _lane_bcast(xia if k < _L else xib, k % _L)
                acc = acc + xk * w_rows[k]
            l_v[i, :] = acc
            rows.append(acc)

        iota = lax.iota(jnp.int32, _L)
        alive = jnp.where((iota >= 1) & (iota < _N), 1, 0)
        path_vec = jnp.zeros((_L,), jnp.int32)
        lcur = rows[0]
        for t in range(_STEPS):
            alive_f = alive.astype(jnp.float32)
            pos = (lax.cumsum(alive_f, axis=0) - alive_f).astype(jnp.int32)
            g = plsc.load_gather(
                blk_v, [jnp.full((_L,), _ROW_G + t, jnp.int32), pos])
            score = jnp.where(alive == 1, lcur + g, -1e30)
            best = jnp.max(score)
            chosen = jnp.min(jnp.where(score == best, iota, 2 * _L))
            chosen_vec = jnp.broadcast_to(chosen, (_L,))
            path_vec = jnp.where(iota == t + 1, chosen_vec, path_vec)
            alive = jnp.where(iota == chosen_vec, 0, alive)
            if t + 1 < _STEPS:
                lcur = plsc.load_gather(l_v, [chosen_vec, iota])
        path_v[...] = path_vec
        pltpu.sync_copy(path_v, out_hbm)


_sample_kernel = functools.partial(
    pl.kernel,
    out_type=jax.ShapeDtypeStruct((_L,), jnp.int32),
    mesh=plsc.VectorSubcoreMesh(
        core_axis_name="c", subcore_axis_name="s",
        num_cores=1, num_subcores=1),
    compiler_params=pltpu.CompilerParams(
        needs_layout_passes=False, skip_device_barrier=True),
    scratch_types=[
        pltpu.VMEM((_ROWS, _L), jnp.float32),
        pltpu.VMEM((_N, _L), jnp.float32),
        pltpu.VMEM((_L,), jnp.int32),
    ],
)(_sample_body)


def _gumbel_table():
    key = jax.random.key(42)
    rows = []
    for m in range(_N - 1, 0, -1):
        key, sk = jax.random.split(key)
        g = jax.random.gumbel(sk, (m,), jnp.float32)
        rows.append(jnp.pad(g, (0, _L - m)))
    return jnp.stack(rows)


def kernel(x, path, W1, b1, W2, b2, W3, b3, Wp, bp):
    del path

    x8 = jnp.zeros((8, _D), jnp.float32).at[:_N].set(x)
    w1t = W1.T
    w2t = W2.T
    w3t = jnp.zeros((_D, 8), jnp.float32).at[:, :1].set(W3.T)
    b1r = b1.reshape(1, 64)
    b2r = b2.reshape(1, 32)
    b3r = jnp.zeros((1, 8), jnp.float32).at[0, 0].set(b3[0])
    energy8 = _energy_head(x8, w1t, b1r, w2t, b2r, w3t, b3r)
    energy = energy8[:_N, 0]

    wpt = jnp.zeros((_D, _L), jnp.float32).at[:, :_N].set(Wp.T)
    blk = jnp.concatenate(
        [
            x.reshape(12, _L),
            wpt,
            jnp.pad(bp, (0, _L - _N)).reshape(1, _L),
            _gumbel_table(),
        ],
        axis=0,
    )
    path16 = _sample_kernel(blk)
    path_indices = path16[:_N + 1]

    return (energy, path_indices)


# speedup vs baseline: 1.0030x; 1.0030x over previous
import functools

import jax
import jax.numpy as jnp
from jax import lax
from jax.experimental import pallas as pl
from jax.experimental.pallas import tpu as pltpu
from jax.experimental.pallas import tpu_sc as plsc

_N = 6
_D = 32
_L = 16
_STEPS = _N - 1

_ROW_X = 0
_ROW_WPT = 12
_ROW_BP = 44
_ROW_G = 45
_ROWS = 50


def _energy_body(x_ref, w1_ref, b1_ref, w2_ref, b2_ref, w3_ref, b3_ref, o_ref):
    x = x_ref[...]
    h = jnp.dot(x, w1_ref[...], preferred_element_type=jnp.float32)
    h = jnp.maximum(h + b1_ref[...], 0.0)
    h = jnp.dot(h, w2_ref[...], preferred_element_type=jnp.float32)
    h = jnp.maximum(h + b2_ref[...], 0.0)
    z = jnp.dot(h, w3_ref[...], preferred_element_type=jnp.float32)
    z = z + b3_ref[...]
    o_ref[...] = jnp.maximum(z, 0.0) + jnp.log(1.0 + jnp.exp(-jnp.abs(z)))


def _energy_head(x8, w1t, b1r, w2t, b2r, w3t, b3r):
    return pl.pallas_call(
        _energy_body,
        out_shape=jax.ShapeDtypeStruct((8, 8), jnp.float32),
    )(x8, w1t, b1r, w2t, b2r, w3t, b3r)


def _lane_bcast(vec, lane):
    idx = jnp.broadcast_to(jnp.asarray(lane, jnp.int32), (_L,))
    return jnp.take_along_axis(vec, idx, axis=0, mode="promise_in_bounds")


def _sample_body(blk_hbm, out_hbm, blk_v, l_v, path_v):
    cid = lax.axis_index("c")
    sid = lax.axis_index("s")

    @pl.when(jnp.logical_and(cid == 0, sid == 0))
    def _():
        pltpu.sync_copy(blk_hbm, blk_v)

        bp_vec = blk_v[_ROW_BP, :]
        def mac_step(k, accs):
            w = blk_v[_ROW_WPT + k, :]
            koff = jnp.where(k >= _L, 1, 0)
            lane = lax.rem(k, _L)
            new = []
            for i in range(_N):
                xrow = blk_v[_ROW_X + 2 * i + koff, :]
                xk = _lane_bcast(xrow, lane)
                new.append(accs[i] + xk * w)
            return tuple(new)

        accs = lax.fori_loop(0, _D, mac_step, (bp_vec,) * _N)
        rows = list(accs)
        for i in range(_N):
            l_v[i, :] = rows[i]

        iota = lax.iota(jnp.int32, _L)
        alive = jnp.where((iota >= 1) & (iota < _N), 1, 0)
        path_vec = jnp.zeros((_L,), jnp.int32)
        lcur = rows[0]
        for t in range(_STEPS):
            alive_f = alive.astype(jnp.float32)
            pos = (lax.cumsum(alive_f, axis=0) - alive_f).astype(jnp.int32)
            g = plsc.load_gather(
                blk_v, [jnp.full((_L,), _ROW_G + t, jnp.int32), pos])
            score = jnp.where(alive == 1, lcur + g, -1e30)
            best = jnp.max(score)
            chosen = jnp.min(jnp.where(score == best, iota, 2 * _L))
            chosen_vec = jnp.broadcast_to(chosen, (_L,))
            path_vec = jnp.where(iota == t + 1, chosen_vec, path_vec)
            alive = jnp.where(iota == chosen_vec, 0, alive)
            if t + 1 < _STEPS:
                lcur = plsc.load_gather(l_v, [chosen_vec, iota])
        path_v[...] = path_vec
        pltpu.sync_copy(path_v, out_hbm)


_sample_kernel = functools.partial(
    pl.kernel,
    out_type=jax.ShapeDtypeStruct((_L,), jnp.int32),
    mesh=plsc.VectorSubcoreMesh(
        core_axis_name="c", subcore_axis_name="s",
        num_cores=1, num_subcores=1),
    compiler_params=pltpu.CompilerParams(
        needs_layout_passes=False, skip_device_barrier=True),
    scratch_types=[
        pltpu.VMEM((_ROWS, _L), jnp.float32),
        pltpu.VMEM((_N, _L), jnp.float32),
        pltpu.VMEM((_L,), jnp.int32),
    ],
)(_sample_body)


def _gumbel_table():
    key = jax.random.key(42)
    rows = []
    for m in range(_N - 1, 0, -1):
        key, sk = jax.random.split(key)
        g = jax.random.gumbel(sk, (m,), jnp.float32)
        rows.append(jnp.pad(g, (0, _L - m)))
    return jnp.stack(rows)


def kernel(x, path, W1, b1, W2, b2, W3, b3, Wp, bp):
    del path

    x8 = jnp.zeros((8, _D), jnp.float32).at[:_N].set(x)
    w1t = W1.T
    w2t = W2.T
    w3t = jnp.zeros((_D, 8), jnp.float32).at[:, :1].set(W3.T)
    b1r = b1.reshape(1, 64)
    b2r = b2.reshape(1, 32)
    b3r = jnp.zeros((1, 8), jnp.float32).at[0, 0].set(b3[0])
    energy8 = _energy_head(x8, w1t, b1r, w2t, b2r, w3t, b3r)
    energy = energy8[:_N, 0]

    wpt = jnp.zeros((_D, _L), jnp.float32).at[:, :_N].set(Wp.T)
    blk = jnp.concatenate(
        [
            x.reshape(12, _L),
            wpt,
            jnp.pad(bp, (0, _L - _N)).reshape(1, _L),
            _gumbel_table(),
        ],
        axis=0,
    )
    path16 = _sample_kernel(blk)
    path_indices = path16[:_N + 1]

    return (energy, path_indices)


# speedup vs baseline: 1.0244x; 1.0214x over previous
import functools

import jax
import jax.numpy as jnp
from jax import lax
from jax.experimental import pallas as pl
from jax.experimental.pallas import tpu as pltpu
from jax.experimental.pallas import tpu_sc as plsc

_N = 6
_D = 32
_L = 16
_STEPS = _N - 1

_ROW_X = 0
_ROW_WPT = 12
_ROW_BP = 44
_ROW_G = 45
_ROWS = 50


def _energy_body(x_ref, w1_ref, b1_ref, w2_ref, b2_ref, w3_ref, b3_ref, o_ref):
    x = x_ref[...]
    h = jnp.dot(x, w1_ref[...], preferred_element_type=jnp.float32)
    h = jnp.maximum(h + b1_ref[...], 0.0)
    h = jnp.dot(h, w2_ref[...], preferred_element_type=jnp.float32)
    h = jnp.maximum(h + b2_ref[...], 0.0)
    z = jnp.dot(h, w3_ref[...], preferred_element_type=jnp.float32)
    z = z + b3_ref[...]
    o_ref[...] = jnp.maximum(z, 0.0) + jnp.log(1.0 + jnp.exp(-jnp.abs(z)))


def _energy_head(x8, w1t, b1r, w2t, b2r, w3t, b3r):
    return pl.pallas_call(
        _energy_body,
        out_shape=jax.ShapeDtypeStruct((8, 8), jnp.float32),
    )(x8, w1t, b1r, w2t, b2r, w3t, b3r)


def _lane_bcast(vec, lane):
    idx = jnp.broadcast_to(jnp.asarray(lane, jnp.int32), (_L,))
    return jnp.take_along_axis(vec, idx, axis=0, mode="promise_in_bounds")


def _row(ref, r):
    return ref[pl.ds(pl.multiple_of(r * _L, _L), _L)]


def _sample_body(blk_hbm, out_hbm, blk_v, l_v, path_v):
    cid = lax.axis_index("c")
    sid = lax.axis_index("s")

    @pl.when(jnp.logical_and(cid == 0, sid == 0))
    def _():
        pltpu.sync_copy(blk_hbm, blk_v)

        bp_vec = _row(blk_v, _ROW_BP)
        def mac_step(k, accs):
            w = _row(blk_v, _ROW_WPT + k)
            koff = jnp.where(k >= _L, 1, 0)
            lane = lax.rem(k, _L)
            new = []
            for i in range(_N):
                xrow = _row(blk_v, _ROW_X + 2 * i + koff)
                xk = _lane_bcast(xrow, lane)
                new.append(accs[i] + xk * w)
            return tuple(new)

        accs = lax.fori_loop(0, _D, mac_step, (bp_vec,) * _N)
        rows = list(accs)
        for i in range(_N):
            l_v[pl.ds(i * _L, _L)] = rows[i]

        iota = lax.iota(jnp.int32, _L)
        alive = jnp.where((iota >= 1) & (iota < _N), 1, 0)
        path_vec = jnp.zeros((_L,), jnp.int32)
        lcur = rows[0]
        for t in range(_STEPS):
            alive_f = alive.astype(jnp.float32)
            pos = (lax.cumsum(alive_f, axis=0) - alive_f).astype(jnp.int32)
            g = plsc.load_gather(blk_v, [(_ROW_G + t) * _L + pos])
            score = jnp.where(alive == 1, lcur + g, -1e30)
            best = jnp.max(score)
            chosen = jnp.min(jnp.where(score == best, iota, 2 * _L))
            chosen_vec = jnp.broadcast_to(chosen, (_L,))
            path_vec = jnp.where(iota == t + 1, chosen_vec, path_vec)
            alive = jnp.where(iota == chosen_vec, 0, alive)
            if t + 1 < _STEPS:
                lcur = plsc.load_gather(l_v, [chosen_vec * _L + iota])
        path_v[...] = path_vec
        pltpu.sync_copy(path_v, out_hbm)


_sample_kernel = functools.partial(
    pl.kernel,
    out_type=jax.ShapeDtypeStruct((_L,), jnp.int32),
    mesh=plsc.VectorSubcoreMesh(
        core_axis_name="c", subcore_axis_name="s",
        num_cores=1, num_subcores=1),
    compiler_params=pltpu.CompilerParams(
        needs_layout_passes=False, skip_device_barrier=True),
    scratch_types=[
        pltpu.VMEM((_ROWS * _L,), jnp.float32),
        pltpu.VMEM((_N * _L,), jnp.float32),
        pltpu.VMEM((_L,), jnp.int32),
    ],
)(_sample_body)


def _gumbel_table():
    key = jax.random.key(42)
    rows = []
    for m in range(_N - 1, 0, -1):
        key, sk = jax.random.split(key)
        g = jax.random.gumbel(sk, (m,), jnp.float32)
        rows.append(jnp.pad(g, (0, _L - m)))
    return jnp.stack(rows)


def kernel(x, path, W1, b1, W2, b2, W3, b3, Wp, bp):
    del path

    x8 = jnp.zeros((8, _D), jnp.float32).at[:_N].set(x)
    w1t = W1.T
    w2t = W2.T
    w3t = jnp.zeros((_D, 8), jnp.float32).at[:, :1].set(W3.T)
    b1r = b1.reshape(1, 64)
    b2r = b2.reshape(1, 32)
    b3r = jnp.zeros((1, 8), jnp.float32).at[0, 0].set(b3[0])
    energy8 = _energy_head(x8, w1t, b1r, w2t, b2r, w3t, b3r)
    energy = energy8[:_N, 0]

    wpt = jnp.zeros((_D, _L), jnp.float32).at[:, :_N].set(Wp.T)
    blk = jnp.concatenate(
        [
            x.reshape(12, _L),
            wpt,
            jnp.pad(bp, (0, _L - _N)).reshape(1, _L),
            _gumbel_table(),
        ],
        axis=0,
    )
    path16 = _sample_kernel(blk.reshape(-1))
    path_indices = path16[:_N + 1]

    return (energy, path_indices)


# speedup vs baseline: 1.0260x; 1.0015x over previous
import functools

import jax
import jax.numpy as jnp
from jax import lax
from jax.experimental import pallas as pl
from jax.experimental.pallas import tpu as pltpu
from jax.experimental.pallas import tpu_sc as plsc

_N = 6
_D = 32
_L = 16
_STEPS = _N - 1

_ROW_X = 0
_ROW_WPT = 12
_ROW_BP = 44
_ROW_G = 45
_ROWS = 50


def _energy_body(x_ref, w1_ref, b1_ref, w2_ref, b2_ref, w3_ref, b3_ref, o_ref):
    x = x_ref[...]
    h = jnp.dot(x, w1_ref[...], preferred_element_type=jnp.float32)
    h = jnp.maximum(h + b1_ref[...], 0.0)
    h = jnp.dot(h, w2_ref[...], preferred_element_type=jnp.float32)
    h = jnp.maximum(h + b2_ref[...], 0.0)
    z = jnp.dot(h, w3_ref[...], preferred_element_type=jnp.float32)
    z = z + b3_ref[...]
    o_ref[...] = jnp.maximum(z, 0.0) + jnp.log(1.0 + jnp.exp(-jnp.abs(z)))


def _energy_head(x8, w1t, b1r, w2t, b2r, w3t, b3r):
    return pl.pallas_call(
        _energy_body,
        out_shape=jax.ShapeDtypeStruct((8, 8), jnp.float32),
    )(x8, w1t, b1r, w2t, b2r, w3t, b3r)


def _lane_bcast(vec, lane):
    idx = jnp.broadcast_to(jnp.asarray(lane, jnp.int32), (_L,))
    return jnp.take_along_axis(vec, idx, axis=0, mode="promise_in_bounds")


def _row(ref, r):
    return ref[pl.ds(pl.multiple_of(r * _L, _L), _L)]


def _sample_body(blk_hbm, out_hbm, blk_s, blk_v, l_v, path_v):
    cid = lax.axis_index("c")
    sid = lax.axis_index("s")

    @pl.when(jnp.logical_and(cid == 0, sid == 0))
    def _():
        pltpu.sync_copy(blk_hbm, blk_s)
        pltpu.sync_copy(blk_s, blk_v)

        bp_vec = _row(blk_v, _ROW_BP)
        def mac_step(k, accs):
            w = _row(blk_v, _ROW_WPT + k)
            koff = jnp.where(k >= _L, 1, 0)
            lane = lax.rem(k, _L)
            new = []
            for i in range(_N):
                xrow = _row(blk_v, _ROW_X + 2 * i + koff)
                xk = _lane_bcast(xrow, lane)
                new.append(accs[i] + xk * w)
            return tuple(new)

        accs = lax.fori_loop(0, _D, mac_step, (bp_vec,) * _N)
        rows = list(accs)
        for i in range(_N):
            l_v[pl.ds(i * _L, _L)] = rows[i]

        iota = lax.iota(jnp.int32, _L)
        alive = jnp.where((iota >= 1) & (iota < _N), 1, 0)
        path_vec = jnp.zeros((_L,), jnp.int32)
        lcur = rows[0]
        for t in range(_STEPS):
            alive_f = alive.astype(jnp.float32)
            pos = (lax.cumsum(alive_f, axis=0) - alive_f).astype(jnp.int32)
            g = plsc.load_gather(blk_v, [(_ROW_G + t) * _L + pos])
            score = jnp.where(alive == 1, lcur + g, -1e30)
            best = jnp.max(score)
            chosen = jnp.min(jnp.where(score == best, iota, 2 * _L))
            chosen_vec = jnp.broadcast_to(chosen, (_L,))
            path_vec = jnp.where(iota == t + 1, chosen_vec, path_vec)
            alive = jnp.where(iota == chosen_vec, 0, alive)
            if t + 1 < _STEPS:
                lcur = plsc.load_gather(l_v, [chosen_vec * _L + iota])
        path_v[...] = path_vec
        pltpu.sync_copy(path_v, out_hbm)


_sample_kernel = functools.partial(
    pl.kernel,
    out_type=jax.ShapeDtypeStruct((_L,), jnp.int32),
    mesh=plsc.VectorSubcoreMesh(
        core_axis_name="c", subcore_axis_name="s",
        num_cores=1, num_subcores=1),
    compiler_params=pltpu.CompilerParams(
        needs_layout_passes=False, skip_device_barrier=True),
    scratch_types=[
        pltpu.VMEM_SHARED((_ROWS * _L,), jnp.float32),
        pltpu.VMEM((_ROWS * _L,), jnp.float32),
        pltpu.VMEM((_N * _L,), jnp.float32),
        pltpu.VMEM((_L,), jnp.int32),
    ],
)(_sample_body)


def _gumbel_table():
    key = jax.random.key(42)
    rows = []
    for m in range(_N - 1, 0, -1):
        key, sk = jax.random.split(key)
        g = jax.random.gumbel(sk, (m,), jnp.float32)
        rows.append(jnp.pad(g, (0, _L - m)))
    return jnp.stack(rows)


def kernel(x, path, W1, b1, W2, b2, W3, b3, Wp, bp):
    del path

    x8 = jnp.zeros((8, _D), jnp.float32).at[:_N].set(x)
    w1t = W1.T
    w2t = W2.T
    w3t = jnp.zeros((_D, 8), jnp.float32).at[:, :1].set(W3.T)
    b1r = b1.reshape(1, 64)
    b2r = b2.reshape(1, 32)
    b3r = jnp.zeros((1, 8), jnp.float32).at[0, 0].set(b3[0])
    energy8 = _energy_head(x8, w1t, b1r, w2t, b2r, w3t, b3r)
    energy = energy8[:_N, 0]

    wpt = jnp.zeros((_D, _L), jnp.float32).at[:, :_N].set(Wp.T)
    blk = jnp.concatenate(
        [
            x.reshape(12, _L),
            wpt,
            jnp.pad(bp, (0, _L - _N)).reshape(1, _L),
            _gumbel_table(),
        ],
        axis=0,
    )
    path16 = _sample_kernel(blk.reshape(-1))
    path_indices = path16[:_N + 1]

    return (energy, path_indices)


# speedup vs baseline: 3.1968x; 3.1159x over previous
import functools

import jax
import jax.numpy as jnp
import numpy as np
from jax import lax
from jax.experimental import pallas as pl
from jax.experimental.pallas import tpu as pltpu
from jax.experimental.pallas import tpu_sc as plsc

_N = 6
_D = 32
_L = 16
_STEPS = _N - 1

_OFF_X = 0
_OFF_WP = 192
_OFF_BP = 384
_OFF_G = 400
_BLK = 480


def _gumbel_table_host():
    key = jax.random.key(42)
    rows = []
    for m in range(_N - 1, 0, -1):
        key, sk = jax.random.split(key)
        g = np.asarray(jax.random.gumbel(sk, (m,), jnp.float32))
        rows.append(np.pad(g, (0, _L - m)))
    return np.concatenate(rows).astype(np.float32)


_GTAB = _gumbel_table_host()


_DN = (((1,), (1,)), ((), ()))


def _energy_body(x_ref, w1_ref, b1_ref, w2_ref, b2_ref, w3_ref, b3_ref, o_ref):
    x = x_ref[...]
    h = lax.dot_general(x, w1_ref[...], _DN, preferred_element_type=jnp.float32)
    h = jnp.maximum(h + b1_ref[...], 0.0)
    h = lax.dot_general(h, w2_ref[...], _DN, preferred_element_type=jnp.float32)
    h = jnp.maximum(h + b2_ref[...], 0.0)
    z = lax.dot_general(h, w3_ref[...], _DN, preferred_element_type=jnp.float32)
    z = z + b3_ref[0, 0]
    o_ref[...] = jnp.maximum(z, 0.0) + jnp.log(1.0 + jnp.exp(-jnp.abs(z)))


def _energy_head(x, w1, b1r, w2, b2r, w3, b3r):
    return pl.pallas_call(
        _energy_body,
        out_shape=jax.ShapeDtypeStruct((_N, 8), jnp.float32),
    )(x, w1, b1r, w2, b2r, w3, b3r)


def _lane_bcast(vec, lane):
    idx = jnp.broadcast_to(jnp.asarray(lane, jnp.int32), (_L,))
    return jnp.take_along_axis(vec, idx, axis=0, mode="promise_in_bounds")


def _row(ref, r):
    return ref[pl.ds(pl.multiple_of(r * _L, _L), _L)]


def _sample_body(blk_hbm, out_hbm, blk_v, l_v, path_v):
    cid = lax.axis_index("c")
    sid = lax.axis_index("s")

    @pl.when(jnp.logical_and(cid == 0, sid == 0))
    def _():
        pltpu.sync_copy(blk_hbm, blk_v)

        iota = lax.iota(jnp.int32, _L)
        bp_vec = blk_v[pl.ds(_OFF_BP, _L)]
        wp_base = _OFF_WP + jnp.minimum(iota, _N - 1) * _D

        def mac_step(k, accs):
            w = plsc.load_gather(blk_v, [wp_base + k])
            koff = jnp.where(k >= _L, _L, 0)
            lane = lax.rem(k, _L)
            new = []
            for i in range(_N):
                xrow = blk_v[pl.ds(_OFF_X + _D * i + koff, _L)]
                xk = _lane_bcast(xrow, lane)
                new.append(accs[i] + xk * w)
            return tuple(new)

        accs = lax.fori_loop(0, _D, mac_step, (bp_vec,) * _N)
        rows = list(accs)
        for i in range(_N):
            l_v[pl.ds(i * _L, _L)] = rows[i]

        alive = jnp.where((iota >= 1) & (iota < _N), 1, 0)
        path_vec = jnp.zeros((_L,), jnp.int32)
        lcur = rows[0]
        for t in range(_STEPS):
            alive_f = alive.astype(jnp.float32)
            pos = (lax.cumsum(alive_f, axis=0) - alive_f).astype(jnp.int32)
            g = plsc.load_gather(blk_v, [_OFF_G + t * _L + pos])
            score = jnp.where(alive == 1, lcur + g, -1e30)
            best = jnp.max(score)
            chosen = jnp.min(jnp.where(score == best, iota, 2 * _L))
            chosen_vec = jnp.broadcast_to(chosen, (_L,))
            path_vec = jnp.where(iota == t + 1, chosen_vec, path_vec)
            alive = jnp.where(iota == chosen_vec, 0, alive)
            if t + 1 < _STEPS:
                lcur = plsc.load_gather(l_v, [chosen_vec * _L + iota])
        path_v[...] = path_vec
        pltpu.sync_copy(path_v, out_hbm)


_sample_kernel = functools.partial(
    pl.kernel,
    out_type=jax.ShapeDtypeStruct((_L,), jnp.int32),
    mesh=plsc.VectorSubcoreMesh(
        core_axis_name="c", subcore_axis_name="s",
        num_cores=1, num_subcores=1),
    compiler_params=pltpu.CompilerParams(
        needs_layout_passes=False, skip_device_barrier=True),
    scratch_types=[
        pltpu.VMEM((_BLK,), jnp.float32),
        pltpu.VMEM((_N * _L,), jnp.float32),
        pltpu.VMEM((_L,), jnp.int32),
    ],
)(_sample_body)


def kernel(x, path, W1, b1, W2, b2, W3, b3, Wp, bp):
    del path

    w3p = jnp.zeros((8, _D), jnp.float32).at[:1].set(W3)
    energy = _energy_head(
        x, W1, b1.reshape(1, 64), W2, b2.reshape(1, 32), w3p, b3.reshape(1, 1)
    )[:, 0]

    blk = jnp.concatenate([
        x.reshape(-1),
        Wp.reshape(-1),
        jnp.pad(bp, (0, _L - _N)),
        jnp.asarray(_GTAB),
    ])
    path16 = _sample_kernel(blk)
    path_indices = path16[:_N + 1]

    return (energy, path_indices)
